# baseline (speedup 1.0000x reference)
import functools

import jax
import jax.numpy as jnp
from jax.experimental import pallas as pl
from jax.experimental.pallas import tpu as pltpu

D_MODEL_ = 4096
NUM_EXPERTS_ = 64
CAPACITY_FACTOR_ = 1.0
EPS_ = 1e-06
NUM_TOKENS_ = 8192

CHUNKS_ = (128, 128, 256) + (512,) * 14 + (256, 128, 128)
MAXC_ = max(CHUNKS_)
NBUF_ = 4
DEPTH_ = 2


def _route_block(gt):
    mx = jnp.max(gt, axis=0, keepdims=True)
    rows = jax.lax.broadcasted_iota(jnp.int32, gt.shape, 0)
    eq = gt == mx
    first = jnp.min(jnp.where(eq, rows, NUM_EXPERTS_), axis=0, keepdims=True)
    return jnp.where(rows == first, gt, 0.0)


def _router_kernel(x_ref, w_ref, out_ref, buf_ref, denom_ref, sems):
    offs = []
    o = 0
    for c in CHUNKS_:
        offs.append(o)
        o += c

    def start(j):
        c = CHUNKS_[j]
        pltpu.make_async_copy(
            x_ref.at[pl.ds(offs[j], c), :],
            buf_ref.at[j % NBUF_, pl.ds(0, c), :],
            sems.at[j],
        ).start()

    for j in range(DEPTH_ + 1):
        start(j)

    w = w_ref[...]
    capacity = jnp.float32(int(CAPACITY_FACTOR_ * NUM_TOKENS_))

    for j, c in enumerate(CHUNKS_):
        pltpu.make_async_copy(
            x_ref.at[pl.ds(offs[j], c), :],
            buf_ref.at[j % NBUF_, pl.ds(0, c), :],
            sems.at[j],
        ).wait()
        if j + DEPTH_ + 1 < len(CHUNKS_):
            start(j + DEPTH_ + 1)

        xc = buf_ref[j % NBUF_, pl.ds(0, c), :]
        gt = jax.lax.dot_general(
            w, xc, dimension_numbers=(((1,), (1,)), ((), ())),
            preferred_element_type=jnp.float32)
        m = _route_block(gt)
        out_ref[pl.ds(offs[j], c), :] = m.T
        part = jnp.sum(m, axis=1, keepdims=True)
        if j == 0:
            denom_ref[...] = part
        else:
            denom_ref[...] += part

    scale = capacity / (denom_ref[...] + EPS_)
    out_ref[...] = out_ref[...] * scale.T


@functools.partial(jax.jit)
def kernel(x, W):
    n_tokens = x.shape[0]
    return pl.pallas_call(
        _router_kernel,
        in_specs=[
            pl.BlockSpec(memory_space=pltpu.MemorySpace.HBM),
            pl.BlockSpec(memory_space=pltpu.MemorySpace.VMEM),
        ],
        out_specs=pl.BlockSpec(memory_space=pltpu.MemorySpace.VMEM),
        out_shape=jax.ShapeDtypeStruct((n_tokens, NUM_EXPERTS_), jnp.float32),
        scratch_shapes=[
            pltpu.VMEM((NBUF_, MAXC_, D_MODEL_), jnp.float32),
            pltpu.VMEM((NUM_EXPERTS_, 1), jnp.float32),
            pltpu.SemaphoreType.DMA((len(CHUNKS_),)),
        ],
    )(x, W)
